# Initial kernel scaffold; baseline (speedup 1.0000x reference)
import jax
import jax.numpy as jnp
from jax.experimental import pallas as pl


def kernel(x, edge_index, l1_W, l1_w1, l1_b1, l1_g1, l1_be1, l1_w2, l1_b2, l1_g2, l1_be2, l1_w3, l1_b3, l2_W, l2_w1, l2_b1, l2_g1, l2_be1, l2_w2, l2_b2, l2_g2, l2_be2, l2_w3, l2_b3):
    raise NotImplementedError("write your pallas kernel here")


# speedup vs baseline: 1.9415x; 1.9415x over previous
import functools
import jax
import jax.numpy as jnp
from jax import lax
from jax.experimental import pallas as pl
from jax.experimental.pallas import tpu as pltpu
from jax.experimental.pallas import tpu_sc as plsc

N = 10000
E = 320000
NC = 2
NS = 16
NW = NC * NS
EW = E // NW
C = 80
NCHUNK = EW // C
NP = 10240
RPW = NP // NS

_slope = 0.2


def _lrelu(x):
    return jnp.where(x >= 0, x, _slope * x)


def _tc_matmul(x, w, bn_rows):
    n, k = x.shape
    m = w.shape[1]

    def body(x_ref, w_ref, o_ref):
        o_ref[...] = jnp.dot(x_ref[...], w_ref[...],
                             preferred_element_type=jnp.float32)

    return pl.pallas_call(
        body,
        grid=(n // bn_rows,),
        in_specs=[pl.BlockSpec((bn_rows, k), lambda i: (i, 0)),
                  pl.BlockSpec((k, m), lambda i: (0, 0))],
        out_specs=pl.BlockSpec((bn_rows, m), lambda i: (i, 0)),
        out_shape=jax.ShapeDtypeStruct((n, m), jnp.float32),
    )(x, w)


def _tc_mlp_stage(a, b_in, w, bias, scale, shift, absdiff, preact, be):
    m = w.shape[1]
    d = w.shape[0]

    def body(*refs):
        i = pl.program_id(0)
        if absdiff:
            a_ref, b_ref, w_ref, bias_ref = refs[:4]
            y_ref, s_ref = refs[4:]
            u = jnp.abs(a_ref[...] - b_ref[...])
        elif preact:
            a_ref, w_ref, bias_ref, sc_ref, sh_ref = refs[:5]
            y_ref, s_ref = refs[5:]
            u = _lrelu(a_ref[...] * sc_ref[...] + sh_ref[...])
        else:
            a_ref, w_ref, bias_ref = refs[:3]
            y_ref, s_ref = refs[3:]
            u = a_ref[...]
        y = jnp.dot(u, w_ref[...], preferred_element_type=jnp.float32) \
            + bias_ref[...]
        y_ref[...] = y

        @pl.when(i == 0)
        def _():
            s_ref[...] = jnp.zeros_like(s_ref)

        s_ref[0:1, :] += jnp.sum(y, axis=0, keepdims=True)
        s_ref[1:2, :] += jnp.sum(y * y, axis=0, keepdims=True)

    ins = [a]
    in_specs = [pl.BlockSpec((be, d), lambda i: (i, 0))]
    if absdiff:
        ins.append(b_in)
        in_specs.append(pl.BlockSpec((be, d), lambda i: (i, 0)))
    ins += [w, bias.reshape(1, m)]
    in_specs += [pl.BlockSpec((d, m), lambda i: (0, 0)),
                 pl.BlockSpec((1, m), lambda i: (0, 0))]
    if preact:
        ins += [scale.reshape(1, d), shift.reshape(1, d)]
        in_specs += [pl.BlockSpec((1, d), lambda i: (0, 0)),
                     pl.BlockSpec((1, d), lambda i: (0, 0))]

    y, s = pl.pallas_call(
        body,
        grid=(E // be,),
        in_specs=in_specs,
        out_specs=[pl.BlockSpec((be, m), lambda i: (i, 0)),
                   pl.BlockSpec((8, m), lambda i: (0, 0))],
        out_shape=[jax.ShapeDtypeStruct((E, m), jnp.float32),
                   jax.ShapeDtypeStruct((8, m), jnp.float32)],
    )(*ins)
    return y, s


def _tc_edge_final(y2, scale, shift, w3, b3, src2, dst2, hg, gidx_is_src,
                   be):
    d = y2.shape[1]
    dg = hg.shape[1]
    nh = dg // 128
    nq = NP // 128

    def body(y_ref, sc_ref, sh_ref, w_ref, b3_ref, s_ref, d_ref, hg_ref,
             *out_refs):
        i = pl.program_id(0)
        z = _lrelu(y_ref[...] * sc_ref[...] + sh_ref[...])
        z3 = jnp.dot(z, w_ref[...], preferred_element_type=jnp.float32) \
            + b3_ref[...]
        z3 = _lrelu(z3)
        sd = (s_ref[...] == d_ref[...]).astype(jnp.float32)
        e = jnp.exp(-z3) + sd
        out_refs[0][...] = e * hg_ref[:, 0:128]
        if nh == 2:
            out_refs[1][...] = e * hg_ref[:, 128:256]

        rs_ref = out_refs[-1]
        gidx = s_ref[...] if gidx_is_src else d_ref[...]
        q = gidx // 128
        r = gidx % 128
        ohq = (q == lax.broadcasted_iota(jnp.int32, (1, nq), 1)) \
            .astype(jnp.float32)
        ohr = (r == lax.broadcasted_iota(jnp.int32, (1, 128), 1)) \
            .astype(jnp.float32)
        contrib = lax.dot_general(ohq, ohr * e, (((0,), (0,)), ((), ())),
                                  precision=lax.Precision.HIGHEST,
                                  preferred_element_type=jnp.float32)

        @pl.when(i == 0)
        def _():
            rs_ref[...] = jnp.zeros_like(rs_ref)

        rs_ref[...] += contrib

    out_specs = [pl.BlockSpec((be, 128), lambda i: (i, 0))]
    out_shape = [jax.ShapeDtypeStruct((E, 128), jnp.float32)]
    if nh == 2:
        out_specs.append(pl.BlockSpec((be, 128), lambda i: (i, 0)))
        out_shape.append(jax.ShapeDtypeStruct((E, 128), jnp.float32))
    out_specs.append(pl.BlockSpec((nq, 128), lambda i: (0, 0)))
    out_shape.append(jax.ShapeDtypeStruct((nq, 128), jnp.float32))

    return pl.pallas_call(
        body,
        grid=(E // be,),
        in_specs=[pl.BlockSpec((be, d), lambda i: (i, 0)),
                  pl.BlockSpec((1, d), lambda i: (0, 0)),
                  pl.BlockSpec((1, d), lambda i: (0, 0)),
                  pl.BlockSpec((d, 1), lambda i: (0, 0)),
                  pl.BlockSpec((1, 1), lambda i: (0, 0)),
                  pl.BlockSpec((be, 1), lambda i: (i, 0)),
                  pl.BlockSpec((be, 1), lambda i: (i, 0)),
                  pl.BlockSpec((be, dg), lambda i: (i, 0))],
        out_specs=out_specs,
        out_shape=out_shape,
    )(y2, scale.reshape(1, d), shift.reshape(1, d), w3, b3.reshape(1, 1),
      src2, dst2, hg)


def _sc_gather2(h, src, dst):
    d = h.shape[1]
    mesh = plsc.VectorSubcoreMesh(core_axis_name="c", subcore_axis_name="s")

    @functools.partial(
        pl.kernel, mesh=mesh,
        out_type=[jax.ShapeDtypeStruct((E, d), jnp.float32),
                  jax.ShapeDtypeStruct((E, d), jnp.float32)],
        scratch_types=[
            pltpu.VMEM((C,), jnp.int32),
            pltpu.VMEM((C,), jnp.int32),
            pltpu.VMEM((C, d), jnp.float32),
            pltpu.VMEM((C, d), jnp.float32),
            pltpu.SemaphoreType.DMA,
        ],
    )
    def k(h_hbm, src_hbm, dst_hbm, hs_hbm, hd_hbm,
          idx_s, idx_d, rows_s, rows_d, sem):
        wid = lax.axis_index("s") * NC + lax.axis_index("c")
        base = wid * EW

        def chunk(c, _):
            off = base + c * C
            pltpu.sync_copy(src_hbm.at[pl.ds(off, C)], idx_s)
            pltpu.sync_copy(dst_hbm.at[pl.ds(off, C)], idx_d)
            cp1 = pltpu.async_copy(h_hbm.at[idx_s], rows_s, sem)
            cp2 = pltpu.async_copy(h_hbm.at[idx_d], rows_d, sem)
            cp1.wait()
            cp2.wait()
            pltpu.sync_copy(rows_s, hs_hbm.at[pl.ds(off, C)])
            pltpu.sync_copy(rows_d, hd_hbm.at[pl.ds(off, C)])
            return ()

        lax.fori_loop(0, NCHUNK, chunk, ())

    return k(h, src, dst)


def _make_sc_scatter(f):
    mesh = plsc.VectorSubcoreMesh(core_axis_name="c", subcore_axis_name="s")

    @functools.partial(
        pl.kernel, mesh=mesh,
        out_type=[jax.ShapeDtypeStruct((NC, NP, f), jnp.float32)],
        scratch_types=[
            pltpu.VMEM((C,), jnp.int32),
            pltpu.VMEM((C, f), jnp.float32),
            pltpu.VMEM_SHARED((NP, f), jnp.float32),
        ],
    )
    def k(rows_hbm, gidx_hbm, zero_hbm, acc_hbm, gidx_v, rows_v, acc_sh):
        cid = lax.axis_index("c")
        sid = lax.axis_index("s")
        wid = sid * NC + cid
        base = wid * EW

        pltpu.sync_copy(zero_hbm.at[pl.ds(sid * RPW, RPW)],
                        acc_sh.at[pl.ds(sid * RPW, RPW)])
        plsc.subcore_barrier()

        def chunk(c, _):
            off = base + c * C
            pltpu.sync_copy(gidx_hbm.at[pl.ds(off, C)], gidx_v)
            pltpu.sync_copy(rows_hbm.at[pl.ds(off, C)], rows_v)
            pltpu.sync_copy(rows_v, acc_sh.at[gidx_v], add=True)
            return ()

        lax.fori_loop(0, NCHUNK, chunk, ())
        plsc.subcore_barrier()

        pltpu.sync_copy(acc_sh.at[pl.ds(sid * RPW, RPW)],
                        acc_hbm.at[cid, pl.ds(sid * RPW, RPW)])

    return k


def _sc_scatter(rows, gidx):
    f = rows.shape[1]
    k = _make_sc_scatter(f)
    zeros = jnp.zeros((NP, f), jnp.float32)
    out = k(rows, gidx, zeros)
    if isinstance(out, (list, tuple)):
        out = out[0]
    return out[:, :N]


def _tc_rowsum(rowsum_vec):
    def body(p_ref, o_ref):
        rowsum = p_ref[...]
        nz = rowsum != 0.0
        has_nz = jnp.any(nz)
        mask = jnp.where(nz, 0.0, 1.0)
        row = lax.broadcasted_iota(jnp.int32, mask.shape, 0)
        mask = jnp.where((row == 0) & has_nz, 0.0, mask)
        o_ref[...] = rowsum + mask

    return pl.pallas_call(
        body,
        out_shape=jax.ShapeDtypeStruct((N, 1), jnp.float32),
    )(rowsum_vec)


def _tc_combine(accs, rs, relu, final_norm, bn_rows):
    nh = len(accs)
    d_out = 128 * nh

    def body(*refs):
        acc_refs = refs[:nh]
        rs_ref = refs[nh]
        o_ref = refs[nh + 1]
        r = rs_ref[...]
        cols = []
        for a_ref in acc_refs:
            hp = (a_ref[0, :, 0:128] + a_ref[1, :, 0:128]) / r
            cols.append(hp)
        h = cols[0] if nh == 1 else jnp.concatenate(cols, axis=1)
        if relu:
            h = _lrelu(h)
        if final_norm:
            nrm = jnp.sqrt(jnp.sum(h * h, axis=1, keepdims=True))
            h = h / jnp.maximum(nrm, 1e-12)
        o_ref[...] = h

    in_specs = [pl.BlockSpec((NC, bn_rows, a.shape[2]), lambda i: (0, i, 0))
                for a in accs]
    in_specs.append(pl.BlockSpec((bn_rows, 1), lambda i: (i, 0)))
    return pl.pallas_call(
        body,
        grid=(N // bn_rows,),
        in_specs=in_specs,
        out_specs=pl.BlockSpec((bn_rows, d_out), lambda i: (i, 0)),
        out_shape=jax.ShapeDtypeStruct((N, d_out), jnp.float32),
    )(*accs, rs)


def _bn_affine(stats, gamma, beta, eps=1e-5):
    s, ss = stats[0], stats[1]
    mean = s / E
    var = ss / E - mean * mean
    scale = gamma / jnp.sqrt(var + eps)
    shift = beta - mean * scale
    return scale, shift


def _gat_layer_opt(x_in, src, dst, gidx, gather_dst, src2, dst2, p, relu,
                   final_norm):
    d = p['W'].shape[1]
    h = _tc_matmul(x_in, p['W'], 2000)
    hs, hd = _sc_gather2(h, src, dst)
    y1, st1 = _tc_mlp_stage(hs, hd, p['w1'], p['b1'], None, None,
                            True, False, 2000)
    sc1, sh1 = _bn_affine(st1, p['g1'], p['be1'])
    y2, st2 = _tc_mlp_stage(y1, None, p['w2'], p['b2'], sc1, sh1,
                            False, True, 4000)
    sc2, sh2 = _bn_affine(st2, p['g2'], p['be2'])
    hg = hd if gather_dst else hs
    scaled = _tc_edge_final(y2, sc2, sh2, p['w3'], p['b3'][0], src2, dst2,
                            hg, gather_dst, 4000)
    rs_mat = scaled[-1]
    accs = [_sc_scatter(scaled[0], gidx)]
    if d == 256:
        accs.append(_sc_scatter(scaled[1], gidx))

    rs = _tc_rowsum(rs_mat.reshape(NP)[:N].reshape(N, 1))
    return _tc_combine(accs, rs, relu, final_norm, 2000)


def kernel(x, edge_index,
           l1_W, l1_w1, l1_b1, l1_g1, l1_be1, l1_w2, l1_b2, l1_g2, l1_be2,
           l1_w3, l1_b3,
           l2_W, l2_w1, l2_b1, l2_g1, l2_be1, l2_w2, l2_b2, l2_g2, l2_be2,
           l2_w3, l2_b3):
    p1 = {'W': l1_W, 'w1': l1_w1, 'b1': l1_b1, 'g1': l1_g1, 'be1': l1_be1,
          'w2': l1_w2, 'b2': l1_b2, 'g2': l1_g2, 'be2': l1_be2,
          'w3': l1_w3, 'b3': l1_b3}
    p2 = {'W': l2_W, 'w1': l2_w1, 'b1': l2_b1, 'g1': l2_g1, 'be1': l2_be1,
          'w2': l2_w2, 'b2': l2_b2, 'g2': l2_g2, 'be2': l2_be2,
          'w3': l2_w3, 'b3': l2_b3}
    src = edge_index[0]
    dst = edge_index[1]
    src2 = src.reshape(E, 1)
    dst2 = dst.reshape(E, 1)
    h = _gat_layer_opt(x, src, dst, src, True, src2, dst2, p1, True, False)
    h = _gat_layer_opt(h, src, dst, dst, False, src2, dst2, p2, False, True)
    return h


# speedup vs baseline: 2.5435x; 1.3101x over previous
import functools
import jax
import jax.numpy as jnp
from jax import lax
from jax.experimental import pallas as pl
from jax.experimental.pallas import tpu as pltpu
from jax.experimental.pallas import tpu_sc as plsc

N = 10000
E = 320000
NC = 2
NS = 16
NW = NC * NS
EW = E // NW
C = 80
NCHUNK = EW // C
NP = 10240
RPW = NP // NS

_slope = 0.2


def _lrelu(x):
    return jnp.where(x >= 0, x, _slope * x)


def _tc_matmul(x, w, bn_rows):
    n, k = x.shape
    m = w.shape[1]

    def body(x_ref, w_ref, o_ref):
        o_ref[...] = jnp.dot(x_ref[...], w_ref[...],
                             preferred_element_type=jnp.float32)

    return pl.pallas_call(
        body,
        grid=(n // bn_rows,),
        in_specs=[pl.BlockSpec((bn_rows, k), lambda i: (i, 0)),
                  pl.BlockSpec((k, m), lambda i: (0, 0))],
        out_specs=pl.BlockSpec((bn_rows, m), lambda i: (i, 0)),
        out_shape=jax.ShapeDtypeStruct((n, m), jnp.float32),
    )(x, w)


def _tc_mlp_stage(a, b_in, w, bias, scale, shift, absdiff, preact, be):
    m = w.shape[1]
    d = w.shape[0]

    def body(*refs):
        i = pl.program_id(0)
        if absdiff:
            a_ref, b_ref, w_ref, bias_ref = refs[:4]
            y_ref, s_ref = refs[4:]
            u = jnp.abs(a_ref[...] - b_ref[...])
        elif preact:
            a_ref, w_ref, bias_ref, sc_ref, sh_ref = refs[:5]
            y_ref, s_ref = refs[5:]
            u = _lrelu(a_ref[...] * sc_ref[...] + sh_ref[...])
        else:
            a_ref, w_ref, bias_ref = refs[:3]
            y_ref, s_ref = refs[3:]
            u = a_ref[...]
        y = jnp.dot(u, w_ref[...], preferred_element_type=jnp.float32) \
            + bias_ref[...]
        y_ref[...] = y

        @pl.when(i == 0)
        def _():
            s_ref[...] = jnp.zeros_like(s_ref)

        s_ref[0:1, :] += jnp.sum(y, axis=0, keepdims=True)
        s_ref[1:2, :] += jnp.sum(y * y, axis=0, keepdims=True)

    ins = [a]
    in_specs = [pl.BlockSpec((be, d), lambda i: (i, 0))]
    if absdiff:
        ins.append(b_in)
        in_specs.append(pl.BlockSpec((be, d), lambda i: (i, 0)))
    ins += [w, bias.reshape(1, m)]
    in_specs += [pl.BlockSpec((d, m), lambda i: (0, 0)),
                 pl.BlockSpec((1, m), lambda i: (0, 0))]
    if preact:
        ins += [scale.reshape(1, d), shift.reshape(1, d)]
        in_specs += [pl.BlockSpec((1, d), lambda i: (0, 0)),
                     pl.BlockSpec((1, d), lambda i: (0, 0))]

    y, s = pl.pallas_call(
        body,
        grid=(E // be,),
        in_specs=in_specs,
        out_specs=[pl.BlockSpec((be, m), lambda i: (i, 0)),
                   pl.BlockSpec((8, m), lambda i: (0, 0))],
        out_shape=[jax.ShapeDtypeStruct((E, m), jnp.float32),
                   jax.ShapeDtypeStruct((8, m), jnp.float32)],
    )(*ins)
    return y, s


def _tc_edge_final(y2, scale, shift, w3, b3, src2, dst2, hg, gidx_is_src,
                   be):
    d = y2.shape[1]
    dg = hg.shape[1]
    nh = dg // 128
    nq = NP // 128

    def body(y_ref, sc_ref, sh_ref, w_ref, b3_ref, s_ref, d_ref, hg_ref,
             *out_refs):
        i = pl.program_id(0)
        z = _lrelu(y_ref[...] * sc_ref[...] + sh_ref[...])
        z3 = jnp.dot(z, w_ref[...], preferred_element_type=jnp.float32) \
            + b3_ref[...]
        z3 = _lrelu(z3)
        sd = (s_ref[...] == d_ref[...]).astype(jnp.float32)
        e = jnp.exp(-z3) + sd
        out_refs[0][...] = e * hg_ref[:, 0:128]
        if nh == 2:
            out_refs[1][...] = e * hg_ref[:, 128:256]

        rs_ref = out_refs[-1]
        gidx = s_ref[...] if gidx_is_src else d_ref[...]
        q = gidx // 128
        r = gidx % 128
        ohq = (q == lax.broadcasted_iota(jnp.int32, (1, nq), 1)) \
            .astype(jnp.bfloat16)
        ohr = (r == lax.broadcasted_iota(jnp.int32, (1, 128), 1)) \
            .astype(jnp.float32)
        contrib = lax.dot_general(ohq, (ohr * e).astype(jnp.bfloat16),
                                  (((0,), (0,)), ((), ())),
                                  preferred_element_type=jnp.float32)

        @pl.when(i == 0)
        def _():
            rs_ref[...] = jnp.zeros_like(rs_ref)

        rs_ref[...] += contrib

    out_specs = [pl.BlockSpec((be, 128), lambda i: (i, 0))]
    out_shape = [jax.ShapeDtypeStruct((E, 128), jnp.float32)]
    if nh == 2:
        out_specs.append(pl.BlockSpec((be, 128), lambda i: (i, 0)))
        out_shape.append(jax.ShapeDtypeStruct((E, 128), jnp.float32))
    out_specs.append(pl.BlockSpec((nq, 128), lambda i: (0, 0)))
    out_shape.append(jax.ShapeDtypeStruct((nq, 128), jnp.float32))

    return pl.pallas_call(
        body,
        grid=(E // be,),
        in_specs=[pl.BlockSpec((be, d), lambda i: (i, 0)),
                  pl.BlockSpec((1, d), lambda i: (0, 0)),
                  pl.BlockSpec((1, d), lambda i: (0, 0)),
                  pl.BlockSpec((d, 1), lambda i: (0, 0)),
                  pl.BlockSpec((1, 1), lambda i: (0, 0)),
                  pl.BlockSpec((be, 1), lambda i: (i, 0)),
                  pl.BlockSpec((be, 1), lambda i: (i, 0)),
                  pl.BlockSpec((be, dg), lambda i: (i, 0))],
        out_specs=out_specs,
        out_shape=out_shape,
    )(y2, scale.reshape(1, d), shift.reshape(1, d), w3, b3.reshape(1, 1),
      src2, dst2, hg)


def _sc_gather2(h, src, dst):
    d = h.shape[1]
    mesh = plsc.VectorSubcoreMesh(core_axis_name="c", subcore_axis_name="s")

    @functools.partial(
        pl.kernel, mesh=mesh,
        out_type=[jax.ShapeDtypeStruct((E, d), jnp.float32),
                  jax.ShapeDtypeStruct((E, d), jnp.float32)],
        scratch_types=[
            pltpu.VMEM((NCHUNK, C), jnp.int32),
            pltpu.VMEM((NCHUNK, C), jnp.int32),
            pltpu.VMEM((2, C, d), jnp.float32),
            pltpu.VMEM((2, C, d), jnp.float32),
            pltpu.SemaphoreType.DMA,
            pltpu.SemaphoreType.DMA,
            pltpu.SemaphoreType.DMA,
            pltpu.SemaphoreType.DMA,
        ],
    )
    def k(h_hbm, src_hbm, dst_hbm, hs_hbm, hd_hbm,
          idx_s, idx_d, rows_s, rows_d, g0, g1, w0, w1):
        wid = lax.axis_index("s") * NC + lax.axis_index("c")
        base = wid * EW
        sem_g = (g0, g1)
        sem_w = (w0, w1)

        pltpu.sync_copy(src_hbm.at[wid], idx_s)
        pltpu.sync_copy(dst_hbm.at[wid], idx_d)

        def fire(c, b):
            pltpu.async_copy(h_hbm.at[idx_s.at[c]], rows_s.at[b], sem_g[b])
            pltpu.async_copy(h_hbm.at[idx_d.at[c]], rows_d.at[b], sem_g[b])

        def wait_gather(b):
            pltpu.make_async_copy(h_hbm.at[idx_s.at[0]], rows_s.at[b],
                                  sem_g[b]).wait()
            pltpu.make_async_copy(h_hbm.at[idx_d.at[0]], rows_d.at[b],
                                  sem_g[b]).wait()

        def writeback(c, b):
            off = base + c * C
            pltpu.async_copy(rows_s.at[b], hs_hbm.at[pl.ds(off, C)],
                             sem_w[b])
            pltpu.async_copy(rows_d.at[b], hd_hbm.at[pl.ds(off, C)],
                             sem_w[b])

        def wait_write(b):
            pltpu.make_async_copy(rows_s.at[b], hs_hbm.at[pl.ds(base, C)],
                                  sem_w[b]).wait()
            pltpu.make_async_copy(rows_d.at[b], hd_hbm.at[pl.ds(base, C)],
                                  sem_w[b]).wait()

        fire(0, 0)

        def pair(t, _):
            c0 = 2 * t

            @pl.when(t >= 1)
            def _():
                wait_write(1)

            fire(c0 + 1, 1)
            wait_gather(0)
            writeback(c0, 0)
            wait_write(0)
            fire(c0 + 2, 0)
            wait_gather(1)
            writeback(c0 + 1, 1)
            return ()

        lax.fori_loop(0, NCHUNK // 2, pair, ())
        wait_write(1)
        wait_gather(0)
        writeback(NCHUNK - 1, 0)
        wait_write(0)

    return k(h, src.reshape(NW, NCHUNK, C), dst.reshape(NW, NCHUNK, C))


def _make_sc_scatter(f):
    mesh = plsc.VectorSubcoreMesh(core_axis_name="c", subcore_axis_name="s")

    @functools.partial(
        pl.kernel, mesh=mesh,
        out_type=[jax.ShapeDtypeStruct((NC, NP, f), jnp.float32)],
        scratch_types=[
            pltpu.VMEM((NCHUNK, C), jnp.int32),
            pltpu.VMEM((2, C, f), jnp.float32),
            pltpu.VMEM_SHARED((NP, f), jnp.float32),
            pltpu.SemaphoreType.DMA,
            pltpu.SemaphoreType.DMA,
        ],
    )
    def k(rows_hbm, gidx_hbm, zero_hbm, acc_hbm, gidx_v, rows_v, acc_sh,
          r0, r1):
        cid = lax.axis_index("c")
        sid = lax.axis_index("s")
        wid = sid * NC + cid
        base = wid * EW
        sem_r = (r0, r1)

        pltpu.sync_copy(zero_hbm.at[pl.ds(sid * RPW, RPW)],
                        acc_sh.at[pl.ds(sid * RPW, RPW)])
        pltpu.sync_copy(gidx_hbm.at[wid], gidx_v)
        plsc.subcore_barrier()

        def fire(c, b):
            off = base + c * C
            pltpu.async_copy(rows_hbm.at[pl.ds(off, C)], rows_v.at[b],
                             sem_r[b])

        def scat(c, b):
            pltpu.make_async_copy(rows_hbm.at[pl.ds(base, C)],
                                  rows_v.at[b], sem_r[b]).wait()
            pltpu.sync_copy(rows_v.at[b], acc_sh.at[gidx_v.at[c]],
                            add=True)

        fire(0, 0)

        def pair(t, _):
            c0 = 2 * t
            fire(c0 + 1, 1)
            scat(c0, 0)
            fire(c0 + 2, 0)
            scat(c0 + 1, 1)
            return ()

        lax.fori_loop(0, NCHUNK // 2, pair, ())
        scat(NCHUNK - 1, 0)
        plsc.subcore_barrier()

        pltpu.sync_copy(acc_sh.at[pl.ds(sid * RPW, RPW)],
                        acc_hbm.at[cid, pl.ds(sid * RPW, RPW)])

    return k


def _sc_scatter(rows, gidx):
    f = rows.shape[1]
    k = _make_sc_scatter(f)
    zeros = jnp.zeros((NP, f), jnp.float32)
    out = k(rows, gidx.reshape(NW, NCHUNK, C), zeros)
    if isinstance(out, (list, tuple)):
        out = out[0]
    return out[:, :N]


def _tc_rowsum(rowsum_vec):
    def body(p_ref, o_ref):
        rowsum = p_ref[...]
        nz = rowsum != 0.0
        has_nz = jnp.any(nz)
        mask = jnp.where(nz, 0.0, 1.0)
        row = lax.broadcasted_iota(jnp.int32, mask.shape, 0)
        mask = jnp.where((row == 0) & has_nz, 0.0, mask)
        o_ref[...] = rowsum + mask

    return pl.pallas_call(
        body,
        out_shape=jax.ShapeDtypeStruct((N, 1), jnp.float32),
    )(rowsum_vec)


def _tc_combine(accs, rs, relu, final_norm, bn_rows):
    nh = len(accs)
    d_out = 128 * nh

    def body(*refs):
        acc_refs = refs[:nh]
        rs_ref = refs[nh]
        o_ref = refs[nh + 1]
        r = rs_ref[...]
        cols = []
        for a_ref in acc_refs:
            hp = (a_ref[0, :, 0:128] + a_ref[1, :, 0:128]) / r
            cols.append(hp)
        h = cols[0] if nh == 1 else jnp.concatenate(cols, axis=1)
        if relu:
            h = _lrelu(h)
        if final_norm:
            nrm = jnp.sqrt(jnp.sum(h * h, axis=1, keepdims=True))
            h = h / jnp.maximum(nrm, 1e-12)
        o_ref[...] = h

    in_specs = [pl.BlockSpec((NC, bn_rows, a.shape[2]), lambda i: (0, i, 0))
                for a in accs]
    in_specs.append(pl.BlockSpec((bn_rows, 1), lambda i: (i, 0)))
    return pl.pallas_call(
        body,
        grid=(N // bn_rows,),
        in_specs=in_specs,
        out_specs=pl.BlockSpec((bn_rows, d_out), lambda i: (i, 0)),
        out_shape=jax.ShapeDtypeStruct((N, d_out), jnp.float32),
    )(*accs, rs)


def _bn_affine(stats, gamma, beta, eps=1e-5):
    s, ss = stats[0], stats[1]
    mean = s / E
    var = ss / E - mean * mean
    scale = gamma / jnp.sqrt(var + eps)
    shift = beta - mean * scale
    return scale, shift


def _gat_layer_opt(x_in, src, dst, gidx, gather_dst, src2, dst2, p, relu,
                   final_norm):
    d = p['W'].shape[1]
    h = _tc_matmul(x_in, p['W'], 2000)
    hs, hd = _sc_gather2(h, src, dst)
    y1, st1 = _tc_mlp_stage(hs, hd, p['w1'], p['b1'], None, None,
                            True, False, 2000)
    sc1, sh1 = _bn_affine(st1, p['g1'], p['be1'])
    y2, st2 = _tc_mlp_stage(y1, None, p['w2'], p['b2'], sc1, sh1,
                            False, True, 4000)
    sc2, sh2 = _bn_affine(st2, p['g2'], p['be2'])
    hg = hd if gather_dst else hs
    scaled = _tc_edge_final(y2, sc2, sh2, p['w3'], p['b3'][0], src2, dst2,
                            hg, gather_dst, 4000)
    rs_mat = scaled[-1]
    accs = [_sc_scatter(scaled[0], gidx)]
    if d == 256:
        accs.append(_sc_scatter(scaled[1], gidx))

    rs = _tc_rowsum(rs_mat.reshape(NP)[:N].reshape(N, 1))
    return _tc_combine(accs, rs, relu, final_norm, 2000)


def kernel(x, edge_index,
           l1_W, l1_w1, l1_b1, l1_g1, l1_be1, l1_w2, l1_b2, l1_g2, l1_be2,
           l1_w3, l1_b3,
           l2_W, l2_w1, l2_b1, l2_g1, l2_be1, l2_w2, l2_b2, l2_g2, l2_be2,
           l2_w3, l2_b3):
    p1 = {'W': l1_W, 'w1': l1_w1, 'b1': l1_b1, 'g1': l1_g1, 'be1': l1_be1,
          'w2': l1_w2, 'b2': l1_b2, 'g2': l1_g2, 'be2': l1_be2,
          'w3': l1_w3, 'b3': l1_b3}
    p2 = {'W': l2_W, 'w1': l2_w1, 'b1': l2_b1, 'g1': l2_g1, 'be1': l2_be1,
          'w2': l2_w2, 'b2': l2_b2, 'g2': l2_g2, 'be2': l2_be2,
          'w3': l2_w3, 'b3': l2_b3}
    src = edge_index[0]
    dst = edge_index[1]
    src2 = src.reshape(E, 1)
    dst2 = dst.reshape(E, 1)
    h = _gat_layer_opt(x, src, dst, src, True, src2, dst2, p1, True, False)
    h = _gat_layer_opt(h, src, dst, dst, False, src2, dst2, p2, False, True)
    return h
